# baseline (speedup 1.0000x reference)
import functools

import jax
import jax.numpy as jnp
from jax import lax
from jax.experimental import pallas as pl
from jax.experimental.pallas import tpu as pltpu
from jax.experimental.pallas import tpu_sc as plsc

N = 16384
C = 1000
NC = 2
NS = 16
NW = NC * NS
RPW = N // NW
CHUNK = 64
NCHUNK = RPW // CHUNK
L = 16

_mesh = plsc.VectorSubcoreMesh(core_axis_name="c", subcore_axis_name="s")


@functools.partial(
    pl.kernel,
    out_type=jax.ShapeDtypeStruct((N * C,), jnp.float32),
    mesh=_mesh,
    scratch_types=[
        pltpu.VMEM((RPW,), jnp.int32),
        pltpu.VMEM((CHUNK * C,), jnp.float32),
    ],
    compiler_params=pltpu.CompilerParams(needs_layout_passes=False),
)
def _onehot_sc(x_hbm, z_hbm, out_hbm, idx_v, buf_v):
    wid = lax.axis_index("s") * NC + lax.axis_index("c")
    base = wid * RPW
    pltpu.sync_copy(x_hbm.at[pl.ds(base, RPW)], idx_v)
    pltpu.sync_copy(z_hbm, buf_v)

    ones = jnp.full((L,), 1.0, jnp.float32)
    zeros = jnp.zeros((L,), jnp.float32)
    rows = lax.iota(jnp.int32, L)

    def chunk_body(k, carry):
        for j in range(CHUNK // L):
            xv = idx_v[pl.ds(k * CHUNK + j * L, L)]
            offs = (rows + j * L) * C + xv
            plsc.store_scatter(buf_v, [offs], ones)
        pltpu.sync_copy(
            buf_v, out_hbm.at[pl.ds((base + k * CHUNK) * C, CHUNK * C)]
        )
        for j in range(CHUNK // L):
            xv = idx_v[pl.ds(k * CHUNK + j * L, L)]
            offs = (rows + j * L) * C + xv
            plsc.store_scatter(buf_v, [offs], zeros)
        return carry

    lax.fori_loop(0, NCHUNK, chunk_body, 0)


def kernel(x):
    z = jnp.zeros((CHUNK * C,), jnp.float32)
    flat = _onehot_sc(x.astype(jnp.int32), z)
    return flat.reshape(N, C)


# speedup vs baseline: 1.5531x; 1.5531x over previous
import functools

import jax
import jax.numpy as jnp
from jax import lax
from jax.experimental import pallas as pl
from jax.experimental.pallas import tpu as pltpu
from jax.experimental.pallas import tpu_sc as plsc

N = 16384
C = 1000
NC = 2
NS = 16
NW = NC * NS
RPW = N // NW
CHUNK = 64
NCHUNK = RPW // CHUNK
L = 16

_mesh = plsc.VectorSubcoreMesh(core_axis_name="c", subcore_axis_name="s")


@functools.partial(
    pl.kernel,
    out_type=jax.ShapeDtypeStruct((N, C), jnp.float32),
    mesh=_mesh,
    scratch_types=[
        pltpu.VMEM((RPW,), jnp.int32),
        pltpu.VMEM((CHUNK, C), jnp.float32),
    ],
    compiler_params=pltpu.CompilerParams(needs_layout_passes=False),
)
def _onehot_sc(x_hbm, z_hbm, out_hbm, idx_v, buf_v):
    wid = lax.axis_index("s") * NC + lax.axis_index("c")
    base = wid * RPW
    pltpu.sync_copy(x_hbm.at[pl.ds(base, RPW)], idx_v)
    pltpu.sync_copy(z_hbm, buf_v)

    ones = jnp.full((L,), 1.0, jnp.float32)
    zeros = jnp.zeros((L,), jnp.float32)
    rows = lax.iota(jnp.int32, L)

    def chunk_body(k, carry):
        for j in range(CHUNK // L):
            xv = idx_v[pl.ds(k * CHUNK + j * L, L)]
            plsc.store_scatter(buf_v, [rows + j * L, xv], ones)
        pltpu.sync_copy(buf_v, out_hbm.at[pl.ds(base + k * CHUNK, CHUNK)])
        for j in range(CHUNK // L):
            xv = idx_v[pl.ds(k * CHUNK + j * L, L)]
            plsc.store_scatter(buf_v, [rows + j * L, xv], zeros)
        return carry

    lax.fori_loop(0, NCHUNK, chunk_body, 0)


def kernel(x):
    z = jnp.zeros((CHUNK, C), jnp.float32)
    return _onehot_sc(x.astype(jnp.int32), z)


# speedup vs baseline: 3.0561x; 1.9677x over previous
import functools

import jax
import jax.numpy as jnp
from jax import lax
from jax.experimental import pallas as pl
from jax.experimental.pallas import tpu as pltpu
from jax.experimental.pallas import tpu_sc as plsc

N = 16384
C = 1000
NC = 2
NS = 16
NW = NC * NS
CPW = N // NW
BLK = 128
NBLK = CPW // BLK
L = 16

_mesh = plsc.VectorSubcoreMesh(core_axis_name="c", subcore_axis_name="s")


@functools.partial(
    pl.kernel,
    out_type=jax.ShapeDtypeStruct((C, N), jnp.float32),
    mesh=_mesh,
    scratch_types=[
        pltpu.VMEM((CPW,), jnp.int32),
        pltpu.VMEM((C, BLK), jnp.float32),
    ],
    compiler_params=pltpu.CompilerParams(needs_layout_passes=False),
)
def _onehot_sc(x_hbm, z_hbm, out_hbm, idx_v, buf_v):
    wid = lax.axis_index("s") * NC + lax.axis_index("c")
    base = wid * CPW
    pltpu.sync_copy(x_hbm.at[pl.ds(base, CPW)], idx_v)
    pltpu.sync_copy(z_hbm, buf_v)

    ones = jnp.full((L,), 1.0, jnp.float32)
    zeros = jnp.zeros((L,), jnp.float32)
    cols = lax.iota(jnp.int32, L)

    for b in range(NBLK):
        for j in range(BLK // L):
            xv = idx_v[pl.ds(b * BLK + j * L, L)]
            plsc.store_scatter(buf_v, [xv, cols + j * L], ones)
        pltpu.sync_copy(
            buf_v, out_hbm.at[pl.ds(0, C), pl.ds(base + b * BLK, BLK)]
        )
        if b != NBLK - 1:
            for j in range(BLK // L):
                xv = idx_v[pl.ds(b * BLK + j * L, L)]
                plsc.store_scatter(buf_v, [xv, cols + j * L], zeros)


def kernel(x):
    z = jnp.zeros((C, BLK), jnp.float32)
    return _onehot_sc(x.astype(jnp.int32), z).T


# speedup vs baseline: 3.0669x; 1.0035x over previous
import functools

import jax
import jax.numpy as jnp
from jax import lax
from jax.experimental import pallas as pl
from jax.experimental.pallas import tpu as pltpu
from jax.experimental.pallas import tpu_sc as plsc

N = 16384
C = 1000
CA = 504
CB = C - CA
NC = 2
NS = 16
NW = NC * NS
CPW = N // NW
BLK = 128
NBLK = CPW // BLK
L = 16

_mesh = plsc.VectorSubcoreMesh(core_axis_name="c", subcore_axis_name="s")


@functools.partial(
    pl.kernel,
    out_type=jax.ShapeDtypeStruct((C, N), jnp.float32),
    mesh=_mesh,
    scratch_types=[
        pltpu.VMEM((CPW,), jnp.int32),
        pltpu.VMEM((CA, BLK), jnp.float32),
        pltpu.VMEM((CB, BLK), jnp.float32),
        pltpu.SemaphoreType.DMA,
        pltpu.SemaphoreType.DMA,
    ],
    compiler_params=pltpu.CompilerParams(needs_layout_passes=False),
)
def _onehot_sc(x_hbm, za_hbm, zb_hbm, out_hbm, idx_v, buf_a, buf_b, sem_a, sem_b):
    wid = lax.axis_index("s") * NC + lax.axis_index("c")
    base = wid * CPW
    init_a = pltpu.async_copy(za_hbm, buf_a, sem_a)
    init_b = pltpu.async_copy(zb_hbm, buf_b, sem_b)
    pltpu.sync_copy(x_hbm.at[pl.ds(base, CPW)], idx_v)

    ones = jnp.full((L,), 1.0, jnp.float32)
    zeros = jnp.zeros((L,), jnp.float32)
    cols = lax.iota(jnp.int32, L)

    def scatter(buf, row0, nrows, b, val):
        for j in range(BLK // L):
            xv = idx_v[pl.ds(b * BLK + j * L, L)]
            rv = xv - row0
            mask = (xv >= row0) & (xv < row0 + nrows)
            plsc.store_scatter(buf, [rv, cols + j * L], val, mask=mask)

    prev_a = init_a
    prev_b = init_b
    for b in range(NBLK):
        prev_a.wait()
        if b > 0:
            scatter(buf_a, 0, CA, b - 1, zeros)
        scatter(buf_a, 0, CA, b, ones)
        prev_a = pltpu.async_copy(
            buf_a, out_hbm.at[pl.ds(0, CA), pl.ds(base + b * BLK, BLK)], sem_a
        )
        prev_b.wait()
        if b > 0:
            scatter(buf_b, CA, CB, b - 1, zeros)
        scatter(buf_b, CA, CB, b, ones)
        prev_b = pltpu.async_copy(
            buf_b, out_hbm.at[pl.ds(CA, CB), pl.ds(base + b * BLK, BLK)], sem_b
        )
    prev_a.wait()
    prev_b.wait()


def kernel(x):
    za = jnp.zeros((CA, BLK), jnp.float32)
    zb = jnp.zeros((CB, BLK), jnp.float32)
    return _onehot_sc(x.astype(jnp.int32), za, zb).T
